# baseline (speedup 1.0000x reference)
import functools

import jax
import jax.numpy as jnp
from jax import lax
from jax.experimental import pallas as pl
from jax.experimental.pallas import tpu as pltpu
from jax.experimental.pallas import tpu_sc as plsc

N = 10000
E = 320000
D = 128
D_OUT = 64

NC, NS = 2, 16
NW = NC * NS
EPW = 10240
EPAD = NW * EPW
BE = 64
NB = EPW // BE
SCH = 32
NPAD = 10240
ROWS_PER_TILE = NPAD // NS
DCH = 128
DRPW = EPW // DCH

_sc_mesh = plsc.VectorSubcoreMesh(core_axis_name="c", subcore_axis_name="s")


@functools.partial(
    pl.kernel,
    out_type=jax.ShapeDtypeStruct((NC, NPAD), jnp.float32),
    mesh=_sc_mesh,
    scratch_types=[
        pltpu.VMEM_SHARED((NPAD,), jnp.float32),
        pltpu.VMEM((ROWS_PER_TILE,), jnp.float32),
        pltpu.VMEM((DRPW, DCH), jnp.int32),
        pltpu.VMEM((DRPW, DCH), jnp.float32),
        pltpu.SemaphoreType.DMA,
    ],
)
def _deg_kernel(dst2_hbm, ew2_hbm, out_hbm, acc_sp, zbuf_v, dstb, ewb, dsem):
    cid = lax.axis_index("c")
    sid = lax.axis_index("s")
    wid = sid * NC + cid
    zeros16 = jnp.zeros((16,), jnp.float32)

    @pl.loop(0, ROWS_PER_TILE // 16)
    def _(i):
        zbuf_v[pl.ds(i * 16, 16)] = zeros16

    pltpu.sync_copy(zbuf_v, acc_sp.at[pl.ds(sid * ROWS_PER_TILE, ROWS_PER_TILE)])

    pltpu.sync_copy(dst2_hbm.at[pl.ds(wid * DRPW, DRPW)], dstb)
    pltpu.sync_copy(ew2_hbm.at[pl.ds(wid * DRPW, DRPW)], ewb)
    plsc.subcore_barrier()

    @pl.loop(0, DRPW // 8)
    def _(b):
        @pl.loop(0, 8)
        def _(i):
            j = b * 8 + i
            pltpu.async_copy(ewb.at[j], acc_sp.at[dstb.at[j]], dsem, add=True)

        @pl.loop(0, 8)
        def _(i):
            pltpu.make_async_copy(ewb.at[0], acc_sp.at[dstb.at[0]], dsem).wait()

    plsc.subcore_barrier()
    pltpu.sync_copy(
        acc_sp.at[pl.ds(sid * ROWS_PER_TILE, ROWS_PER_TILE)],
        out_hbm.at[cid, pl.ds(sid * ROWS_PER_TILE, ROWS_PER_TILE)],
    )


@functools.partial(
    pl.kernel,
    out_type=jax.ShapeDtypeStruct((NC, NPAD, D), jnp.float32),
    mesh=_sc_mesh,
    scratch_types=(
        [pltpu.VMEM_SHARED((NPAD, D), jnp.float32)]
        + [pltpu.VMEM((BE, 2 * D), jnp.float32)] * 2
        + [pltpu.VMEM((SCH, D), jnp.float32)] * 2
        + [pltpu.VMEM((BE,), jnp.int32)] * 2
        + [pltpu.VMEM((BE,), jnp.int32)] * 2
        + [pltpu.VMEM((BE,), jnp.float32)] * 2
        + [pltpu.VMEM((SCH,), jnp.int32)] * 8
        + [pltpu.SemaphoreType.DMA] * 5
    ),
)
def _agg_kernel(ys2_hbm, src1_hbm, ew1_hbm, dst1_hbm, out_hbm,
                acc_sp, *bufs_and_sems):
    gb = bufs_and_sems[0:2]
    sbufs = bufs_and_sems[2:4]
    srcv = bufs_and_sems[4:6]
    srcg = bufs_and_sems[6:8]
    ewv = bufs_and_sems[8:10]
    dstr = [bufs_and_sems[10 + 2 * r:12 + 2 * r] for r in range(4)]
    gsem = bufs_and_sems[18:20]
    isem = bufs_and_sems[20:22]
    ssem = bufs_and_sems[22]
    cid = lax.axis_index("c")
    sid = lax.axis_index("s")
    wid = sid * NC + cid
    ebase = wid * EPW
    zeros16 = jnp.zeros((16,), jnp.float32)

    def i_start(h, r, b):
        off = ebase + b * BE
        pltpu.async_copy(src1_hbm.at[pl.ds(off, BE)], srcv[h], isem[h])
        pltpu.async_copy(ew1_hbm.at[pl.ds(off, BE)], ewv[h], isem[h])
        for c in range(2):
            pltpu.async_copy(dst1_hbm.at[pl.ds(off + c * SCH, SCH)],
                             dstr[r][c], isem[h])

    def i_wait(h, r, b):
        off = ebase + b * BE
        pltpu.make_async_copy(src1_hbm.at[pl.ds(off, BE)], srcv[h],
                              isem[h]).wait()
        pltpu.make_async_copy(ew1_hbm.at[pl.ds(off, BE)], ewv[h],
                              isem[h]).wait()
        for c in range(2):
            pltpu.make_async_copy(dst1_hbm.at[pl.ds(off + c * SCH, SCH)],
                                  dstr[r][c], isem[h]).wait()
        @pl.loop(0, BE // 16)
        def _(g):
            srcg[h][pl.ds(g * 16, 16)] = srcv[h][pl.ds(g * 16, 16)] >> 1

    def g_start(h, b):
        for c in range(2):
            pltpu.async_copy(
                ys2_hbm.at[srcg[h].at[pl.ds(c * SCH, SCH)]],
                gb[h].at[pl.ds(c * SCH, SCH)], gsem[h])

    def g_wait(h, b):
        for c in range(2):
            pltpu.make_async_copy(
                ys2_hbm.at[srcg[h].at[pl.ds(c * SCH, SCH)]],
                gb[h].at[pl.ds(c * SCH, SCH)], gsem[h]).wait()

    def s_start(r, s):
        pltpu.async_copy(sbufs[s], acc_sp.at[dstr[r][s]], ssem, add=True)

    def s_wait(r, s):
        pltpu.make_async_copy(sbufs[s], acc_sp.at[dstr[r][s]], ssem).wait()

    def select_scale(h, s):
        gp = gb[h]
        ep = ewv[h]
        sp = srcv[h]
        sb = sbufs[s]

        @pl.loop(0, SCH // 16)
        def _(g):
            ew16 = ep[pl.ds(s * SCH + g * 16, 16)]
            sv16 = sp[pl.ds(s * SCH + g * 16, 16)]
            for k16 in range(16):
                w = ew16[k16]
                par = sv16[k16] & 1
                base = par * D
                k = g * 16 + k16
                for c in range(D // 16):
                    sb[k, pl.ds(c * 16, 16)] = (
                        gp[s * SCH + k, pl.ds(base + c * 16, 16)] * w)

    @pl.loop(0, SCH)
    def _(r):
        for c in range(D // 16):
            sbufs[0][r, pl.ds(c * 16, 16)] = zeros16

    @pl.loop(0, ROWS_PER_TILE // SCH)
    def _(i):
        pltpu.sync_copy(
            sbufs[0], acc_sp.at[pl.ds(sid * ROWS_PER_TILE + i * SCH, SCH)])

    plsc.subcore_barrier()

    i_start(0, 0, 0)
    i_start(1, 1, 1)
    i_wait(0, 0, 0)
    g_start(0, 0)

    @pl.loop(0, NB // 4)
    def _(t):
        for q in range(4):
            h, o, r = q % 2, 1 - q % 2, q
            b = t * 4 + q

            @pl.when(b + 1 < NB)
            def _():
                i_wait(o, (r + 1) % 4, b + 1)
                g_start(o, b + 1)

            g_wait(h, b)

            @pl.when(b >= 1)
            def _():
                for s in range(2):
                    s_wait((r + 3) % 4, s)

            for s in range(2):
                select_scale(h, s)
                s_start(r, s)

            @pl.when(b + 2 < NB)
            def _():
                i_start(h, (r + 2) % 4, b + 2)

    for s in range(2):
        s_wait((NB - 1) % 4, s)

    plsc.subcore_barrier()
    pltpu.sync_copy(
        acc_sp.at[pl.ds(sid * ROWS_PER_TILE, ROWS_PER_TILE)],
        out_hbm.at[cid, pl.ds(sid * ROWS_PER_TILE, ROWS_PER_TILE)],
    )


_BT = 1000


def _k1_body(deg_ref, x_ref, w1_ref, ys_ref, dinv_ref):
    deg = deg_ref[0] + deg_ref[1] + 1.0
    dinv = lax.rsqrt(jnp.maximum(deg, 1e-12))
    y = jnp.dot(x_ref[...], w1_ref[...], preferred_element_type=jnp.float32)
    ys_ref[...] = y * dinv
    dinv_ref[...] = dinv


def _k3_body(acc_ref, ys_ref, dinv_ref, b1_ref, hs_ref):
    dinv = dinv_ref[...]
    t = dinv * (acc_ref[0] + acc_ref[1] + ys_ref[...]) + b1_ref[...]
    hs_ref[...] = jnp.maximum(t, 0.0) * dinv


def _k5_body(acc_ref, hs_ref, dinv_ref, w2_ref, b2_ref, w3_ref, b3_ref,
             mean_ref, logstd_ref):
    u = dinv_ref[...] * (acc_ref[0] + acc_ref[1] + hs_ref[...])
    mean_ref[...] = jnp.dot(u, w2_ref[...],
                            preferred_element_type=jnp.float32) + b2_ref[...]
    logstd_ref[...] = jnp.dot(u, w3_ref[...],
                              preferred_element_type=jnp.float32) + b3_ref[...]


def _row_spec(d):
    return pl.BlockSpec((_BT, d), lambda i: (i, 0))


def _pair_spec(d):
    return pl.BlockSpec((2, _BT, d), lambda i: (0, i, 0))


def _full_spec(a, b):
    return pl.BlockSpec((a, b), lambda i: (0, 0))


def kernel(x, edge_index, edge_weight, W1, b1, W2, b2, W3, b3):
    ei = edge_index.astype(jnp.int32)
    pad_i = jnp.zeros((EPAD - E,), jnp.int32)
    pad_f = jnp.zeros((EPAD - E,), jnp.float32)
    src1 = jnp.concatenate([ei[0], pad_i])
    ew1 = jnp.concatenate([edge_weight, pad_f])
    dst1 = jnp.concatenate([ei[1], pad_i])
    dst2 = dst1.reshape(EPAD // DCH, DCH)
    ew2 = ew1.reshape(EPAD // DCH, DCH)

    deg_parts = _deg_kernel(dst2, ew2)
    deg2 = deg_parts[:, :N, None]

    ys, dinv = pl.pallas_call(
        _k1_body,
        grid=(N // _BT,),
        in_specs=[_pair_spec(1), _row_spec(D), _full_spec(D, D)],
        out_specs=[_row_spec(D), _row_spec(1)],
        out_shape=[jax.ShapeDtypeStruct((N, D), jnp.float32),
                   jax.ShapeDtypeStruct((N, 1), jnp.float32)],
    )(deg2, x, W1)

    acc1 = _agg_kernel(ys.reshape(N // 2, 2 * D), src1, ew1, dst1)[:, :N]

    hs = pl.pallas_call(
        _k3_body,
        grid=(N // _BT,),
        in_specs=[_pair_spec(D), _row_spec(D), _row_spec(1), _full_spec(1, D)],
        out_specs=_row_spec(D),
        out_shape=jax.ShapeDtypeStruct((N, D), jnp.float32),
    )(acc1, ys, dinv, b1.reshape(1, D))

    acc2 = _agg_kernel(hs.reshape(N // 2, 2 * D), src1, ew1, dst1)[:, :N]

    mean, logstd = pl.pallas_call(
        _k5_body,
        grid=(N // _BT,),
        in_specs=[_pair_spec(D), _row_spec(D), _row_spec(1),
                  _full_spec(D, D_OUT), _full_spec(1, D_OUT),
                  _full_spec(D, D_OUT), _full_spec(1, D_OUT)],
        out_specs=[_row_spec(D_OUT), _row_spec(D_OUT)],
        out_shape=[jax.ShapeDtypeStruct((N, D_OUT), jnp.float32),
                   jax.ShapeDtypeStruct((N, D_OUT), jnp.float32)],
    )(acc2, hs, dinv, W2, b2.reshape(1, D_OUT), W3, b3.reshape(1, D_OUT))

    return (mean, logstd)


# speedup vs baseline: 1.3972x; 1.3972x over previous
import functools

import jax
import jax.numpy as jnp
from jax import lax
from jax.experimental import pallas as pl
from jax.experimental.pallas import tpu as pltpu
from jax.experimental.pallas import tpu_sc as plsc

N = 10000
E = 320000
D = 128
D_OUT = 64

NC, NS = 2, 16
NW = NC * NS
EPW = 10240
EPAD = NW * EPW
BE = 160
NB = EPW // BE
SCH = 80
NPAD = 10240
ROWS_PER_TILE = NPAD // NS
DCH = 128
DRPW = EPW // DCH

_sc_mesh = plsc.VectorSubcoreMesh(core_axis_name="c", subcore_axis_name="s")


@functools.partial(
    pl.kernel,
    out_type=jax.ShapeDtypeStruct((NC, NPAD), jnp.float32),
    mesh=_sc_mesh,
    scratch_types=[
        pltpu.VMEM_SHARED((NPAD,), jnp.float32),
        pltpu.VMEM((ROWS_PER_TILE,), jnp.float32),
        pltpu.VMEM((DRPW, DCH), jnp.int32),
        pltpu.VMEM((DRPW, DCH), jnp.float32),
        pltpu.SemaphoreType.DMA,
    ],
)
def _deg_kernel(dst2_hbm, ew2_hbm, out_hbm, acc_sp, zbuf_v, dstb, ewb, dsem):
    cid = lax.axis_index("c")
    sid = lax.axis_index("s")
    wid = sid * NC + cid
    zeros16 = jnp.zeros((16,), jnp.float32)

    @pl.loop(0, ROWS_PER_TILE // 16)
    def _(i):
        zbuf_v[pl.ds(i * 16, 16)] = zeros16

    pltpu.sync_copy(zbuf_v, acc_sp.at[pl.ds(sid * ROWS_PER_TILE, ROWS_PER_TILE)])

    pltpu.sync_copy(dst2_hbm.at[pl.ds(wid * DRPW, DRPW)], dstb)
    pltpu.sync_copy(ew2_hbm.at[pl.ds(wid * DRPW, DRPW)], ewb)
    plsc.subcore_barrier()

    @pl.loop(0, DRPW // 8)
    def _(b):
        @pl.loop(0, 8)
        def _(i):
            j = b * 8 + i
            pltpu.async_copy(ewb.at[j], acc_sp.at[dstb.at[j]], dsem, add=True)

        @pl.loop(0, 8)
        def _(i):
            pltpu.make_async_copy(ewb.at[0], acc_sp.at[dstb.at[0]], dsem).wait()

    plsc.subcore_barrier()
    pltpu.sync_copy(
        acc_sp.at[pl.ds(sid * ROWS_PER_TILE, ROWS_PER_TILE)],
        out_hbm.at[cid, pl.ds(sid * ROWS_PER_TILE, ROWS_PER_TILE)],
    )


@functools.partial(
    pl.kernel,
    out_type=jax.ShapeDtypeStruct((NC, NPAD, D), jnp.float32),
    mesh=_sc_mesh,
    scratch_types=(
        [pltpu.VMEM_SHARED((NPAD, D), jnp.float32)]
        + [pltpu.VMEM((BE, D), jnp.float32)] * 2
        + [pltpu.VMEM((BE,), jnp.int32)] * 2
        + [pltpu.VMEM((BE,), jnp.float32)] * 2
        + [pltpu.VMEM((SCH,), jnp.int32)] * 8
        + [pltpu.SemaphoreType.DMA] * 6
    ),
)
def _agg_kernel(ys_hbm, src1_hbm, ew1_hbm, dst1_hbm, out_hbm,
                acc_sp, *bufs_and_sems):
    rows = bufs_and_sems[0:2]
    srcv = bufs_and_sems[2:4]
    ewv = bufs_and_sems[4:6]
    dstr = [bufs_and_sems[6 + 2 * r:8 + 2 * r] for r in range(4)]
    gsem = bufs_and_sems[14:16]
    ssem = bufs_and_sems[16:18]
    isem = bufs_and_sems[18:20]
    cid = lax.axis_index("c")
    sid = lax.axis_index("s")
    wid = sid * NC + cid
    ebase = wid * EPW
    zeros16 = jnp.zeros((16,), jnp.float32)

    def i_start(h, r, b):
        off = ebase + b * BE
        pltpu.async_copy(src1_hbm.at[pl.ds(off, BE)], srcv[h], isem[h])
        pltpu.async_copy(ew1_hbm.at[pl.ds(off, BE)], ewv[h], isem[h])
        for c in range(2):
            pltpu.async_copy(dst1_hbm.at[pl.ds(off + c * SCH, SCH)],
                             dstr[r][c], isem[h])

    def i_wait(h, r, b):
        off = ebase + b * BE
        pltpu.make_async_copy(src1_hbm.at[pl.ds(off, BE)], srcv[h],
                              isem[h]).wait()
        pltpu.make_async_copy(ew1_hbm.at[pl.ds(off, BE)], ewv[h],
                              isem[h]).wait()
        for c in range(2):
            pltpu.make_async_copy(dst1_hbm.at[pl.ds(off + c * SCH, SCH)],
                                  dstr[r][c], isem[h]).wait()

    def g_start(h, b):
        for c in range(2):
            pltpu.async_copy(
                ys_hbm.at[srcv[h].at[pl.ds(c * SCH, SCH)]],
                rows[h].at[pl.ds(c * SCH, SCH)], gsem[h])

    def g_wait(h, b):
        for c in range(2):
            pltpu.make_async_copy(
                ys_hbm.at[srcv[h].at[pl.ds(c * SCH, SCH)]],
                rows[h].at[pl.ds(c * SCH, SCH)], gsem[h]).wait()

    def s_start(h, r, b):
        for c in range(2):
            pltpu.async_copy(rows[h].at[pl.ds(c * SCH, SCH)],
                             acc_sp.at[dstr[r][c]], ssem[h], add=True)

    def s_wait(h, r, b):
        for c in range(2):
            pltpu.make_async_copy(rows[h].at[pl.ds(c * SCH, SCH)],
                                  acc_sp.at[dstr[r][c]], ssem[h]).wait()

    def scale(h, b):
        rp = rows[h]
        ep = ewv[h]

        @pl.loop(0, BE // 16)
        def _(g):
            ew16 = ep[pl.ds(g * 16, 16)]
            for k16 in range(16):
                w = ew16[k16]
                k = g * 16 + k16
                for c in range(D // 16):
                    rp[k, pl.ds(c * 16, 16)] = rp[k, pl.ds(c * 16, 16)] * w

    @pl.loop(0, BE)
    def _(r):
        for c in range(D // 16):
            rows[0][r, pl.ds(c * 16, 16)] = zeros16

    @pl.loop(0, ROWS_PER_TILE // BE)
    def _(i):
        pltpu.sync_copy(
            rows[0], acc_sp.at[pl.ds(sid * ROWS_PER_TILE + i * BE, BE)])

    plsc.subcore_barrier()

    i_start(0, 0, 0)
    i_start(1, 1, 1)
    i_wait(0, 0, 0)
    g_start(0, 0)

    @pl.loop(0, NB // 4)
    def _(t):
        for q in range(4):
            h, o, r = q % 2, 1 - q % 2, q
            b = t * 4 + q

            @pl.when(b >= 1)
            def _():
                s_wait(o, (r + 3) % 4, b - 1)

            @pl.when(b + 1 < NB)
            def _():
                i_wait(o, (r + 1) % 4, b + 1)
                g_start(o, b + 1)

            g_wait(h, b)
            scale(h, b)
            s_start(h, r, b)

            @pl.when(b + 2 < NB)
            def _():
                i_start(h, (r + 2) % 4, b + 2)

    s_wait((NB - 1) % 2, (NB - 1) % 4, NB - 1)

    plsc.subcore_barrier()
    pltpu.sync_copy(
        acc_sp.at[pl.ds(sid * ROWS_PER_TILE, ROWS_PER_TILE)],
        out_hbm.at[cid, pl.ds(sid * ROWS_PER_TILE, ROWS_PER_TILE)],
    )


_BT = 1000


def _k1_body(deg_ref, x_ref, w1_ref, ys_ref, dinv_ref):
    deg = deg_ref[0] + deg_ref[1] + 1.0
    dinv = lax.rsqrt(jnp.maximum(deg, 1e-12))
    y = jnp.dot(x_ref[...], w1_ref[...], preferred_element_type=jnp.float32)
    ys_ref[...] = y * dinv
    dinv_ref[...] = dinv


def _k3_body(acc_ref, ys_ref, dinv_ref, b1_ref, hs_ref):
    dinv = dinv_ref[...]
    t = dinv * (acc_ref[0] + acc_ref[1] + ys_ref[...]) + b1_ref[...]
    hs_ref[...] = jnp.maximum(t, 0.0) * dinv


def _k5_body(acc_ref, hs_ref, dinv_ref, w2_ref, b2_ref, w3_ref, b3_ref,
             mean_ref, logstd_ref):
    u = dinv_ref[...] * (acc_ref[0] + acc_ref[1] + hs_ref[...])
    mean_ref[...] = jnp.dot(u, w2_ref[...],
                            preferred_element_type=jnp.float32) + b2_ref[...]
    logstd_ref[...] = jnp.dot(u, w3_ref[...],
                              preferred_element_type=jnp.float32) + b3_ref[...]


def _row_spec(d):
    return pl.BlockSpec((_BT, d), lambda i: (i, 0))


def _pair_spec(d):
    return pl.BlockSpec((2, _BT, d), lambda i: (0, i, 0))


def _full_spec(a, b):
    return pl.BlockSpec((a, b), lambda i: (0, 0))


def kernel(x, edge_index, edge_weight, W1, b1, W2, b2, W3, b3):
    ei = edge_index.astype(jnp.int32)
    pad_i = jnp.zeros((EPAD - E,), jnp.int32)
    pad_f = jnp.zeros((EPAD - E,), jnp.float32)
    src1 = jnp.concatenate([ei[0], pad_i])
    ew1 = jnp.concatenate([edge_weight, pad_f])
    dst1 = jnp.concatenate([ei[1], pad_i])
    dst2 = dst1.reshape(EPAD // DCH, DCH)
    ew2 = ew1.reshape(EPAD // DCH, DCH)

    deg_parts = _deg_kernel(dst2, ew2)
    deg2 = deg_parts[:, :N, None]

    ys, dinv = pl.pallas_call(
        _k1_body,
        grid=(N // _BT,),
        in_specs=[_pair_spec(1), _row_spec(D), _full_spec(D, D)],
        out_specs=[_row_spec(D), _row_spec(1)],
        out_shape=[jax.ShapeDtypeStruct((N, D), jnp.float32),
                   jax.ShapeDtypeStruct((N, 1), jnp.float32)],
    )(deg2, x, W1)

    acc1 = _agg_kernel(ys, src1, ew1, dst1)[:, :N]

    hs = pl.pallas_call(
        _k3_body,
        grid=(N // _BT,),
        in_specs=[_pair_spec(D), _row_spec(D), _row_spec(1), _full_spec(1, D)],
        out_specs=_row_spec(D),
        out_shape=jax.ShapeDtypeStruct((N, D), jnp.float32),
    )(acc1, ys, dinv, b1.reshape(1, D))

    acc2 = _agg_kernel(hs, src1, ew1, dst1)[:, :N]

    mean, logstd = pl.pallas_call(
        _k5_body,
        grid=(N // _BT,),
        in_specs=[_pair_spec(D), _row_spec(D), _row_spec(1),
                  _full_spec(D, D_OUT), _full_spec(1, D_OUT),
                  _full_spec(D, D_OUT), _full_spec(1, D_OUT)],
        out_specs=[_row_spec(D_OUT), _row_spec(D_OUT)],
        out_shape=[jax.ShapeDtypeStruct((N, D_OUT), jnp.float32),
                   jax.ShapeDtypeStruct((N, D_OUT), jnp.float32)],
    )(acc2, hs, dinv, W2, b2.reshape(1, D_OUT), W3, b3.reshape(1, D_OUT))

    return (mean, logstd)
